# Initial kernel scaffold; baseline (speedup 1.0000x reference)
import jax
import jax.numpy as jnp
from jax.experimental import pallas as pl


def kernel(boxes, scores, classes):
    raise NotImplementedError("write your pallas kernel here")


# speedup vs baseline: 5.3615x; 5.3615x over previous
import jax
import jax.numpy as jnp
from jax import lax
from jax.experimental import pallas as pl
from jax.experimental.pallas import tpu as pltpu
from jax.experimental.pallas import tpu_sc as plsc

N = 5000
NPAD = 5120
B = 160
NB = NPAD // B
NS = 16
L = 16
VPB = B // L
NMS_THRESH = 0.75


def _sup_mask(px1, py1, px2, py2, pa, x1, y1, x2, y2, ca):
    iw = jnp.maximum(jnp.minimum(px2, x2) - jnp.maximum(px1, x1), 0.0)
    ih = jnp.maximum(jnp.minimum(py2, y2) - jnp.maximum(py1, y1), 0.0)
    inter = iw * ih
    union = jnp.maximum(pa + ca - inter, 1e-9)
    return (inter / union) > NMS_THRESH


def _nms_body(x1h, y1h, x2h, y2h, keep_out,
              x1v, y1v, x2v, y2v, areav, keepv, pub):
    wid = lax.axis_index("s")

    pltpu.sync_copy(x1h, x1v)
    pltpu.sync_copy(y1h, y1v)
    pltpu.sync_copy(x2h, x2v)
    pltpu.sync_copy(y2h, y2v)

    def _init(i, _):
        sl = pl.ds(i * L, L)
        areav[sl] = (x2v[sl] - x1v[sl]) * (y2v[sl] - y1v[sl])
        keepv[sl] = jnp.full((L,), 1.0, jnp.float32)
        return 0
    lax.fori_loop(0, NPAD // L, _init, 0)

    def _load_pivots(pvbase):
        sl = pl.ds(pvbase, L)
        return (x1v[sl], y1v[sl], x2v[sl], y2v[sl], areav[sl])

    def _cross_update(prevbase, cbase):
        def _grp(g, _):
            pvbase = prevbase + g * L
            px1v, py1v, px2v, py2v, pav = _load_pivots(pvbase)
            kgv = keepv[pl.ds(pvbase, L)]
            for i in range(L):
                mk = kgv[i] > 0.0

                @pl.when(mk)
                def _():
                    px1, py1, px2, py2, pa = (px1v[i], py1v[i], px2v[i],
                                              py2v[i], pav[i])

                    def _cand(v, _):
                        sl = pl.ds(cbase + v * L, L)
                        sup = _sup_mask(px1, py1, px2, py2, pa,
                                        x1v[sl], y1v[sl], x2v[sl], y2v[sl],
                                        areav[sl])
                        keepv[sl] = jnp.where(sup, 0.0, keepv[sl])
                        return 0
                    lax.fori_loop(0, VPB, _cand, 0)
            return 0
        lax.fori_loop(0, VPB, _grp, 0)

    def _inblock(cbase):
        lane = lax.iota(jnp.int32, L)

        def _grp(g, _):
            pvbase = cbase + g * L
            px1v, py1v, px2v, py2v, pav = _load_pivots(pvbase)
            kgv = keepv[pl.ds(pvbase, L)]
            for i in range(L):
                mkf = kgv[i]
                mk = mkf > 0.0
                px1, py1, px2, py2, pa = (px1v[i], py1v[i], px2v[i],
                                          py2v[i], pav[i])
                sup_self = _sup_mask(px1, py1, px2, py2, pa,
                                     px1v, py1v, px2v, py2v, pav)
                kgv_sup = jnp.where(lane > i, kgv * (1.0 - mkf), kgv)
                kgv = jnp.where(sup_self, kgv_sup, kgv)

                @pl.when(mk)
                def _():
                    def _cand(v, _):
                        sl = pl.ds(cbase + v * L, L)
                        sup = _sup_mask(px1, py1, px2, py2, pa,
                                        x1v[sl], y1v[sl], x2v[sl], y2v[sl],
                                        areav[sl])
                        keepv[sl] = jnp.where(sup, 0.0, keepv[sl])
                        return 0
                    lax.fori_loop(g + 1, VPB, _cand, 0)
            keepv[pl.ds(pvbase, L)] = kgv
            return 0
        lax.fori_loop(0, VPB, _grp, 0)

    def _round(c, _):
        prev = c - 1

        @pl.when(c > 0)
        def _():
            psl = pl.ds(prev * B, B)
            pltpu.sync_copy(pub.at[psl], keepv.at[psl])
            for ob in (wid, wid + NS):
                @pl.when(ob >= c)
                def _():
                    _cross_update(prev * B, ob * B)

        @pl.when(c % NS == wid)
        def _():
            cbase = c * B
            _inblock(cbase)
            csl = pl.ds(cbase, B)
            pltpu.sync_copy(keepv.at[csl], pub.at[csl])

        plsc.subcore_barrier()
        return 0

    lax.fori_loop(0, NB, _round, 0)

    for ob in (wid, wid + NS):
        osl = pl.ds(ob * B, B)
        pltpu.sync_copy(keepv.at[osl], keep_out.at[osl])


@jax.jit
def _nms_keep(x1, y1, x2, y2):
    mesh = plsc.VectorSubcoreMesh(
        core_axis_name="c", subcore_axis_name="s", num_cores=1)
    f = pl.kernel(
        _nms_body,
        out_type=jax.ShapeDtypeStruct((NPAD,), jnp.float32),
        mesh=mesh,
        scratch_types=[
            pltpu.VMEM((NPAD,), jnp.float32),
            pltpu.VMEM((NPAD,), jnp.float32),
            pltpu.VMEM((NPAD,), jnp.float32),
            pltpu.VMEM((NPAD,), jnp.float32),
            pltpu.VMEM((NPAD,), jnp.float32),
            pltpu.VMEM((NPAD,), jnp.float32),
            pltpu.VMEM_SHARED((NPAD,), jnp.float32),
        ],
    )
    return f(x1, y1, x2, y2)


def kernel(boxes, scores, classes):
    max_coord = jnp.max(boxes) + 1.0
    offsets = classes.astype(boxes.dtype) * max_coord
    boxes_off = boxes + offsets[:, None]
    order = jnp.argsort(-scores)
    b_sorted = jnp.take(boxes_off, order, axis=0)
    b_orig_sorted = jnp.take(boxes, order, axis=0)
    s_sorted = jnp.take(scores, order, axis=0)

    pad = jnp.full((NPAD - N,), -1e6, jnp.float32)
    x1 = jnp.concatenate([b_sorted[:, 0], pad])
    y1 = jnp.concatenate([b_sorted[:, 1], pad])
    x2 = jnp.concatenate([b_sorted[:, 2], pad])
    y2 = jnp.concatenate([b_sorted[:, 3], pad])

    keepf = _nms_keep(x1, y1, x2, y2)[:N]
    out = jnp.concatenate(
        [b_orig_sorted * keepf[:, None], (s_sorted * keepf)[:, None]], axis=1)
    return out


# speedup vs baseline: 21.0629x; 3.9286x over previous
import jax
import jax.numpy as jnp
from jax import lax
from jax.experimental import pallas as pl
from jax.experimental.pallas import tpu as pltpu
from jax.experimental.pallas import tpu_sc as plsc

N = 5000
NPAD = 5120
B = 160
NB = NPAD // B
NS = 16
L = 16
VPB = B // L
HV = VPB // 2
NMS_THRESH = 0.75


def _nms_body(x1h, y1h, x2h, y2h, keep_out,
              x1v, y1v, x2v, y2v, areav, keepv, pub):
    wid = lax.axis_index("s")

    pltpu.sync_copy(x1h, x1v)
    pltpu.sync_copy(y1h, y1v)
    pltpu.sync_copy(x2h, x2v)
    pltpu.sync_copy(y2h, y2v)

    def _init(i, _):
        sl = pl.ds(i * L, L)
        areav[sl] = (x2v[sl] - x1v[sl]) * (y2v[sl] - y1v[sl])
        keepv[sl] = jnp.full((L,), 1.0, jnp.float32)
        return 0
    lax.fori_loop(0, NPAD // L, _init, 0)

    def _load_pivots(pvbase):
        sl = pl.ds(pvbase, L)
        return (x1v[sl], y1v[sl], x2v[sl], y2v[sl], areav[sl])

    def _pair_update(px1, py1, px2, py2, pa, omk,
                     cx1, cy1, cx2, cy2, ca, kc):
        iw = jnp.maximum(jnp.minimum(px2, cx2) - jnp.maximum(px1, cx1), 0.0)
        ih = jnp.maximum(jnp.minimum(py2, cy2) - jnp.maximum(py1, cy1), 0.0)
        inter = iw * ih
        sup = inter > NMS_THRESH * (pa + ca - inter)
        return jnp.where(sup, kc * omk, kc)

    def _cross_update(pbase, cbase):
        for half in range(2):
            hbase = cbase + half * (HV * L)
            sls = [pl.ds(hbase + k * L, L) for k in range(HV)]
            cx1 = [x1v[s] for s in sls]
            cy1 = [y1v[s] for s in sls]
            cx2 = [x2v[s] for s in sls]
            cy2 = [y2v[s] for s in sls]
            ca = [areav[s] for s in sls]
            kc0 = tuple(keepv[s] for s in sls)

            def _pg(g, kcs):
                pvbase = pbase + g * L
                px1v, py1v, px2v, py2v, pav = _load_pivots(pvbase)
                kgv = keepv[pl.ds(pvbase, L)]
                kcs = list(kcs)
                for i in range(L):
                    omk = 1.0 - kgv[i]
                    px1, py1, px2, py2, pa = (px1v[i], py1v[i], px2v[i],
                                              py2v[i], pav[i])
                    for k in range(HV):
                        kcs[k] = _pair_update(px1, py1, px2, py2, pa, omk,
                                              cx1[k], cy1[k], cx2[k], cy2[k],
                                              ca[k], kcs[k])
                return tuple(kcs)

            kcs = lax.fori_loop(0, VPB, _pg, kc0)
            for k in range(HV):
                keepv[sls[k]] = kcs[k]

    def _inblock(cbase):
        lane = lax.iota(jnp.int32, L)

        def _pg(g, _):
            pvbase = cbase + g * L
            px1v, py1v, px2v, py2v, pav = _load_pivots(pvbase)
            kgv = keepv[pl.ds(pvbase, L)]
            for i in range(L):
                omk = 1.0 - kgv[i]
                px1, py1, px2, py2, pa = (px1v[i], py1v[i], px2v[i],
                                          py2v[i], pav[i])
                iw = jnp.maximum(
                    jnp.minimum(px2, px2v) - jnp.maximum(px1, px1v), 0.0)
                ih = jnp.maximum(
                    jnp.minimum(py2, py2v) - jnp.maximum(py1, py1v), 0.0)
                inter = iw * ih
                sup = inter > NMS_THRESH * (pa + pav - inter)
                kg_sup = jnp.where(lane > i, kgv * omk, kgv)
                kgv = jnp.where(sup, kg_sup, kgv)
            keepv[pl.ds(pvbase, L)] = kgv

            def _dv(v, _):
                sl = pl.ds(cbase + v * L, L)
                cx1, cy1, cx2, cy2 = x1v[sl], y1v[sl], x2v[sl], y2v[sl]
                ca = areav[sl]
                kc = keepv[sl]
                for i in range(L):
                    omk = 1.0 - kgv[i]
                    kc = _pair_update(px1v[i], py1v[i], px2v[i], py2v[i],
                                      pav[i], omk, cx1, cy1, cx2, cy2, ca, kc)
                keepv[sl] = kc
                return 0
            lax.fori_loop(g + 1, VPB, _dv, 0)
            return 0
        lax.fori_loop(0, VPB, _pg, 0)

    def _round(c, _):
        prev = c - 1

        @pl.when(c > 0)
        def _():
            psl = pl.ds(prev * B, B)
            pltpu.sync_copy(pub.at[psl], keepv.at[psl])

            def _own(k, _):
                ob = jnp.where(k == 0, wid, (NB - 1) - wid)

                @pl.when(ob >= c)
                def _():
                    _cross_update(prev * B, ob * B)
                return 0
            lax.fori_loop(0, 2, _own, 0)

        @pl.when(jnp.minimum(c, (NB - 1) - c) == wid)
        def _():
            cbase = c * B
            _inblock(cbase)
            csl = pl.ds(cbase, B)
            pltpu.sync_copy(keepv.at[csl], pub.at[csl])

        plsc.subcore_barrier()
        return 0

    lax.fori_loop(0, NB, _round, 0)

    for ob in (wid, (NB - 1) - wid):
        osl = pl.ds(ob * B, B)
        pltpu.sync_copy(keepv.at[osl], keep_out.at[osl])


@jax.jit
def _nms_keep(x1, y1, x2, y2):
    mesh = plsc.VectorSubcoreMesh(
        core_axis_name="c", subcore_axis_name="s", num_cores=1)
    f = pl.kernel(
        _nms_body,
        out_type=jax.ShapeDtypeStruct((NPAD,), jnp.float32),
        mesh=mesh,
        scratch_types=[
            pltpu.VMEM((NPAD,), jnp.float32),
            pltpu.VMEM((NPAD,), jnp.float32),
            pltpu.VMEM((NPAD,), jnp.float32),
            pltpu.VMEM((NPAD,), jnp.float32),
            pltpu.VMEM((NPAD,), jnp.float32),
            pltpu.VMEM((NPAD,), jnp.float32),
            pltpu.VMEM_SHARED((NPAD,), jnp.float32),
        ],
    )
    return f(x1, y1, x2, y2)


def kernel(boxes, scores, classes):
    max_coord = jnp.max(boxes) + 1.0
    offsets = classes.astype(boxes.dtype) * max_coord
    boxes_off = boxes + offsets[:, None]
    order = jnp.argsort(-scores)
    b_sorted = jnp.take(boxes_off, order, axis=0)
    b_orig_sorted = jnp.take(boxes, order, axis=0)
    s_sorted = jnp.take(scores, order, axis=0)

    pad = jnp.full((NPAD - N,), -1e6, jnp.float32)
    x1 = jnp.concatenate([b_sorted[:, 0], pad])
    y1 = jnp.concatenate([b_sorted[:, 1], pad])
    x2 = jnp.concatenate([b_sorted[:, 2], pad])
    y2 = jnp.concatenate([b_sorted[:, 3], pad])

    keepf = _nms_keep(x1, y1, x2, y2)[:N]
    out = jnp.concatenate(
        [b_orig_sorted * keepf[:, None], (s_sorted * keepf)[:, None]], axis=1)
    return out
